# baseline (speedup 1.0000x reference)
import functools

import jax
import jax.numpy as jnp
from jax import lax
from jax.experimental import pallas as pl
from jax.experimental.pallas import tpu as pltpu
from jax.experimental.pallas import tpu_sc as plsc

K = 64
N = 4096
D = 512
B = 8
NPAD = 72
NROW = 2 * B


def _row_cumsum(x):
    n = x.shape[1]
    sh = 1
    while sh < n:
        x = x + jnp.pad(x, ((0, 0), (sh, 0)))[:, :n]
        sh *= 2
    return x


def _topk_body(s_ref, vals_ref, ids_ref):
    s = s_ref[...]
    raw = lax.bitcast_convert_type(s, jnp.int32)
    key = jnp.where(raw >= 0, raw, raw ^ jnp.int32(0x7FFFFFFF))

    lo0 = jnp.full((NROW, 1), jnp.int32(-2147483648))
    hi0 = jnp.full((NROW, 1), jnp.int32(2147483647))

    def bis(_, carry):
        lo, hi = carry
        mid = (lo >> 1) + (hi >> 1) + (lo & hi & 1)
        cnt = jnp.sum((key >= mid).astype(jnp.int32), axis=1, keepdims=True)
        pred = cnt >= K
        return jnp.where(pred, mid, lo), jnp.where(pred, hi, mid)

    T, _ = lax.fori_loop(0, 32, bis, (lo0, hi0))

    gt = key > T
    eq = key == T
    c_gt = jnp.sum(gt.astype(jnp.int32), axis=1, keepdims=True)
    tie_rank = _row_cumsum(eq.astype(jnp.int32)) - eq.astype(jnp.int32)
    sel = gt | (eq & (tie_rank < (K - c_gt)))
    seli = sel.astype(jnp.int32)
    dest = _row_cumsum(seli) - seli

    w = N // 4
    kio = lax.broadcasted_iota(jnp.int32, (NROW, K, w), 1)
    vals64 = jnp.zeros((NROW, K), jnp.float32)
    cols64 = jnp.zeros((NROW, K), jnp.int32)
    for c in range(4):
        vs = s[:, c * w:(c + 1) * w]
        ds = dest[:, c * w:(c + 1) * w]
        ss = sel[:, c * w:(c + 1) * w]
        ind = (ss[:, None, :] & (ds[:, None, :] == kio)).astype(jnp.float32)
        coli = (lax.broadcasted_iota(jnp.int32, (NROW, w), 1) + jnp.int32(c * w))
        vals64 = vals64 + jnp.sum(ind * vs[:, None, :], axis=2)
        cols64 = cols64 + jnp.sum(ind * coli.astype(jnp.float32)[:, None, :],
                                  axis=2).astype(jnp.int32)

    colk = lax.broadcasted_iota(jnp.int32, (NROW, K), 1)

    def step(k, carry):
        v, sv, si = carry
        m = jnp.max(v, axis=1, keepdims=True)
        candc = jnp.where(v >= m, cols64, jnp.int32(1 << 30))
        cmin = jnp.min(candc, axis=1, keepdims=True)
        sv = jnp.where(colk == k, m, sv)
        si = jnp.where(colk == k, cmin, si)
        v = jnp.where((v >= m) & (cols64 == cmin), -jnp.inf, v)
        return v, sv, si

    sv0 = jnp.zeros((NROW, K), jnp.float32)
    si0 = jnp.zeros((NROW, K), jnp.int32)
    _, sv, sids = lax.fori_loop(0, K, step, (vals64, sv0, si0))
    vals_ref[...] = sv
    row = lax.broadcasted_iota(jnp.int32, (NROW, K), 0)
    ids_ref[...] = sids + (row % B) * N


def _topk(scores_stacked):
    return pl.pallas_call(
        _topk_body,
        out_shape=[
            jax.ShapeDtypeStruct((NROW, K), jnp.float32),
            jax.ShapeDtypeStruct((NROW, K), jnp.int32),
        ],
    )(scores_stacked)


def _sc_gather(table, idx_flat):
    n_idx = NROW * K
    nw = 32
    bpw = n_idx // nw
    mesh = plsc.VectorSubcoreMesh(core_axis_name="c", subcore_axis_name="s")

    @functools.partial(
        pl.kernel,
        mesh=mesh,
        out_type=jax.ShapeDtypeStruct((n_idx, D), jnp.float32),
        scratch_types=[
            pltpu.VMEM((bpw,), jnp.int32),
            pltpu.VMEM((bpw, D), jnp.float32),
            pltpu.SemaphoreType.DMA,
        ],
    )
    def gk(table_hbm, idx_hbm, out_hbm, idx_v, rows_v, sem):
        wid = lax.axis_index("s") * 2 + lax.axis_index("c")
        base = wid * bpw
        pltpu.sync_copy(idx_hbm.at[pl.ds(base, bpw)], idx_v)
        pltpu.async_copy(table_hbm.at[idx_v], rows_v, sem).wait()
        pltpu.sync_copy(rows_v, out_hbm.at[pl.ds(base, bpw)])

    return gk(table, idx_flat)


def _mlp_body(ra_ref, ro_ref, na_ref, no_ref, w1_ref, b1_ref, wc_ref, bc_ref,
              cat_ref, sent_ref):
    ra = ra_ref[0]
    ro = ro_ref[0]
    w1a = w1_ref[:D, :]
    w1b = w1_ref[D:, :]
    pa = jnp.dot(ra, w1a, preferred_element_type=jnp.float32)
    pan = jnp.dot(na_ref[0], w1a, preferred_element_type=jnp.float32)
    o65 = jnp.concatenate([ro, no_ref[0]], axis=0)
    po = jnp.dot(o65, w1b, preferred_element_type=jnp.float32)
    po = po + b1_ref[...]
    wc = wc_ref[...]
    bc = bc_ref[...]
    for i in range(K):
        h = jax.nn.relu(pa[i:i + 1, :] + po)
        out = jnp.dot(h, wc, preferred_element_type=jnp.float32) + bc
        cat_ref[0, i * 65:(i + 1) * 65, :] = out[:, :13]
        sent_ref[0, i * 65:(i + 1) * 65, :] = out[:, 13:16]
    h = jax.nn.relu(pan + po[:K, :])
    out = jnp.dot(h, wc, preferred_element_type=jnp.float32) + bc
    cat_ref[0, K * 65:K * 65 + K, :] = out[:, :13]
    sent_ref[0, K * 65:K * 65 + K, :] = out[:, 13:16]


def _mlp(rows16, null_asp, null_opn, w1, b1r, wc, bcr):
    npairs = (K + 1) * (K + 1) - 1
    return pl.pallas_call(
        _mlp_body,
        grid=(B,),
        in_specs=[
            pl.BlockSpec((1, K, D), lambda b: (b, 0, 0)),
            pl.BlockSpec((1, K, D), lambda b: (b + B, 0, 0)),
            pl.BlockSpec((1, 1, D), lambda b: (b, 0, 0)),
            pl.BlockSpec((1, 1, D), lambda b: (b, 0, 0)),
            pl.BlockSpec((2 * D, D), lambda b: (0, 0)),
            pl.BlockSpec((1, D), lambda b: (0, 0)),
            pl.BlockSpec((D, 16), lambda b: (0, 0)),
            pl.BlockSpec((1, 16), lambda b: (0, 0)),
        ],
        out_specs=[
            pl.BlockSpec((1, npairs, 13), lambda b: (b, 0, 0)),
            pl.BlockSpec((1, npairs, 3), lambda b: (b, 0, 0)),
        ],
        out_shape=[
            jax.ShapeDtypeStruct((B, npairs, 13), jnp.float32),
            jax.ShapeDtypeStruct((B, npairs, 3), jnp.float32),
        ],
    )(rows16, rows16, null_asp.reshape(B, 1, D), null_opn.reshape(B, 1, D),
      w1, b1r, wc, bcr)


def kernel(asp_scores, opn_scores, span_reprs, null_asp_repr, null_opn_repr,
           W1, b1, W_cat, b_cat, W_sent, b_sent):
    scores = jnp.concatenate([asp_scores, opn_scores], axis=0)
    vals, ids = _topk(scores)
    asp_topk_scores = vals[:B]
    opn_topk_scores = vals[B:]
    table = span_reprs.reshape(B * N, D)
    rows = _sc_gather(table, ids.reshape(-1))
    rows16 = rows.reshape(2 * B, K, D)

    wc = jnp.concatenate([W_cat, W_sent], axis=1)
    bc = jnp.concatenate([b_cat, b_sent], axis=0).reshape(1, 16)
    cat_logits, sent_logits = _mlp(rows16, null_asp_repr, null_opn_repr,
                                   W1, b1.reshape(1, D), wc, bc)
    return asp_topk_scores, opn_topk_scores, cat_logits, sent_logits


# speedup vs baseline: 1.4559x; 1.4559x over previous
import functools

import jax
import jax.numpy as jnp
from jax import lax
from jax.experimental import pallas as pl
from jax.experimental.pallas import tpu as pltpu
from jax.experimental.pallas import tpu_sc as plsc

K = 64
N = 4096
D = 512
B = 8
NPAD = 72
NROW = 2 * B


def _row_cumsum(x):
    n = x.shape[1]
    sh = 1
    while sh < n:
        x = x + jnp.pad(x, ((0, 0), (sh, 0)))[:, :n]
        sh *= 2
    return x


def _topk_body(s_ref, vals_ref, ids_ref):
    sc0 = s_ref[...]
    col = lax.broadcasted_iota(jnp.int32, (NROW, N), 1)
    lane_k = lax.broadcasted_iota(jnp.int32, (NROW, K), 1)

    def step(k, carry):
        sc, vals, ids = carry
        f = jnp.maximum(sc[:, :N // 2], sc[:, N // 2:])
        f = jnp.maximum(f[:, :N // 4], f[:, N // 4:])
        f = jnp.maximum(f[:, :N // 8], f[:, N // 8:])
        m = jnp.max(f, axis=1, keepdims=True)
        cand = jnp.where(sc >= m, col, jnp.int32(N))
        idx = jnp.min(cand, axis=1, keepdims=True)
        vals = jnp.where(lane_k == k, m, vals)
        ids = jnp.where(lane_k == k, idx, ids)
        sc = jnp.where(col == idx, -jnp.inf, sc)
        return sc, vals, ids

    vals0 = jnp.zeros((NROW, K), jnp.float32)
    ids0 = jnp.zeros((NROW, K), jnp.int32)
    _, vals, ids = lax.fori_loop(0, K, step, (sc0, vals0, ids0))
    vals_ref[...] = vals
    row = lax.broadcasted_iota(jnp.int32, (NROW, K), 0)
    ids_ref[...] = ids + (row % B) * N


def _topk(scores_stacked):
    return pl.pallas_call(
        _topk_body,
        out_shape=[
            jax.ShapeDtypeStruct((NROW, K), jnp.float32),
            jax.ShapeDtypeStruct((NROW, K), jnp.int32),
        ],
    )(scores_stacked)


def _sc_gather(table, idx_flat):
    n_idx = NROW * K
    nw = 32
    bpw = n_idx // nw
    mesh = plsc.VectorSubcoreMesh(core_axis_name="c", subcore_axis_name="s")

    @functools.partial(
        pl.kernel,
        mesh=mesh,
        out_type=jax.ShapeDtypeStruct((n_idx, D), jnp.float32),
        scratch_types=[
            pltpu.VMEM((bpw,), jnp.int32),
            pltpu.VMEM((bpw, D), jnp.float32),
            pltpu.SemaphoreType.DMA,
        ],
    )
    def gk(table_hbm, idx_hbm, out_hbm, idx_v, rows_v, sem):
        wid = lax.axis_index("s") * 2 + lax.axis_index("c")
        base = wid * bpw
        pltpu.sync_copy(idx_hbm.at[pl.ds(base, bpw)], idx_v)
        pltpu.async_copy(table_hbm.at[idx_v], rows_v, sem).wait()
        pltpu.sync_copy(rows_v, out_hbm.at[pl.ds(base, bpw)])

    return gk(table, idx_flat)


def _mlp_body(ra_ref, ro_ref, na_ref, no_ref, w1_ref, b1_ref, wc_ref, bc_ref,
              cat_ref, sent_ref):
    ra = ra_ref[0]
    ro = ro_ref[0]
    w1a = w1_ref[:D, :]
    w1b = w1_ref[D:, :]
    pa = jnp.dot(ra, w1a, preferred_element_type=jnp.float32)
    pan = jnp.dot(na_ref[0], w1a, preferred_element_type=jnp.float32)
    o65 = jnp.concatenate([ro, no_ref[0]], axis=0)
    po = jnp.dot(o65, w1b, preferred_element_type=jnp.float32)
    po = po + b1_ref[...]
    wc = wc_ref[...]
    bc = bc_ref[...]
    for i in range(K):
        h = jax.nn.relu(pa[i:i + 1, :] + po)
        out = jnp.dot(h, wc, preferred_element_type=jnp.float32) + bc
        cat_ref[0, i * 65:(i + 1) * 65, :] = out[:, :13]
        sent_ref[0, i * 65:(i + 1) * 65, :] = out[:, 13:16]
    h = jax.nn.relu(pan + po[:K, :])
    out = jnp.dot(h, wc, preferred_element_type=jnp.float32) + bc
    cat_ref[0, K * 65:K * 65 + K, :] = out[:, :13]
    sent_ref[0, K * 65:K * 65 + K, :] = out[:, 13:16]


def _mlp(rows16, null_asp, null_opn, w1, b1r, wc, bcr):
    npairs = (K + 1) * (K + 1) - 1
    return pl.pallas_call(
        _mlp_body,
        grid=(B,),
        in_specs=[
            pl.BlockSpec((1, K, D), lambda b: (b, 0, 0)),
            pl.BlockSpec((1, K, D), lambda b: (b + B, 0, 0)),
            pl.BlockSpec((1, 1, D), lambda b: (b, 0, 0)),
            pl.BlockSpec((1, 1, D), lambda b: (b, 0, 0)),
            pl.BlockSpec((2 * D, D), lambda b: (0, 0)),
            pl.BlockSpec((1, D), lambda b: (0, 0)),
            pl.BlockSpec((D, 16), lambda b: (0, 0)),
            pl.BlockSpec((1, 16), lambda b: (0, 0)),
        ],
        out_specs=[
            pl.BlockSpec((1, npairs, 13), lambda b: (b, 0, 0)),
            pl.BlockSpec((1, npairs, 3), lambda b: (b, 0, 0)),
        ],
        out_shape=[
            jax.ShapeDtypeStruct((B, npairs, 13), jnp.float32),
            jax.ShapeDtypeStruct((B, npairs, 3), jnp.float32),
        ],
    )(rows16, rows16, null_asp.reshape(B, 1, D), null_opn.reshape(B, 1, D),
      w1, b1r, wc, bcr)


def kernel(asp_scores, opn_scores, span_reprs, null_asp_repr, null_opn_repr,
           W1, b1, W_cat, b_cat, W_sent, b_sent):
    scores = jnp.concatenate([asp_scores, opn_scores], axis=0)
    vals, ids = _topk(scores)
    asp_topk_scores = vals[:B]
    opn_topk_scores = vals[B:]
    table = span_reprs.reshape(B * N, D)
    rows = _sc_gather(table, ids.reshape(-1))
    rows16 = rows.reshape(2 * B, K, D)

    wc = jnp.concatenate([W_cat, W_sent], axis=1)
    bc = jnp.concatenate([b_cat, b_sent], axis=0).reshape(1, 16)
    cat_logits, sent_logits = _mlp(rows16, null_asp_repr, null_opn_repr,
                                   W1, b1.reshape(1, D), wc, bc)
    return asp_topk_scores, opn_topk_scores, cat_logits, sent_logits
